# baseline (speedup 1.0000x reference)
import jax
import jax.numpy as jnp
from jax.experimental import pallas as pl
from jax.experimental.pallas import tpu as pltpu

B, N, C = 8, 576, 1024
NK = N + 1
LEFT = 72
CLUSTER_K = 32
BB = 2


def _kernel(q_ref, k_ref, x_ref, out_ref):
    f32 = jnp.float32
    i32 = jnp.int32
    bf16 = jnp.bfloat16
    scale = f32(C) ** f32(-0.5)
    ones_c = jnp.ones((1, C), dtype=f32)
    ones_nk_bf = jnp.ones((1, NK), dtype=bf16)
    io_c = jax.lax.broadcasted_iota(i32, (NK, NK), 0)
    io_r = jax.lax.broadcasted_iota(i32, (NK, NK), 1)
    tri = io_c < io_r
    lane_row = jax.lax.broadcasted_iota(i32, (1, NK), 1)
    rank_rows = jax.lax.broadcasted_iota(i32, (LEFT, NK), 0).astype(f32)
    iota2 = jax.lax.broadcasted_iota(i32, (LEFT, NK), 1)

    per_b = []
    skeys = []
    for bb in range(BB):
        q2 = q_ref[bb, 0:1, :]
        kf = k_ref[bb]

        lk = jax.lax.dot_general(q2, kf, (((1,), (1,)), ((), ())),
                                 preferred_element_type=f32) * scale
        m = jnp.max(lk)
        ek = jnp.exp(lk - m)
        attn = ek / jnp.sum(ek)
        a_rank = jnp.where(lane_row == 0, f32(-1.0), attn)
        a_col = jnp.transpose(a_rank)

        beats = ((a_col > a_rank) | ((a_col == a_rank) & tri)).astype(bf16)
        rank = jax.lax.dot_general(ones_nk_bf, beats, (((1,), (0,)), ((), ())),
                                   preferred_element_type=f32)
        P = (rank == rank_rows).astype(f32)
        S = (rank < f32(LEFT)).astype(f32)

        nsq = jax.lax.dot_general(ones_c, kf * kf, (((1,), (1,)), ((), ())),
                                  preferred_element_type=f32)
        invn = 1.0 / jnp.maximum(jnp.sqrt(nsq), f32(1e-12))

        ksel = jax.lax.dot_general(P, kf, (((1,), (0,)), ((), ())),
                                   preferred_element_type=f32)
        invnsel = jax.lax.dot_general(P, invn, (((1,), (1,)), ((), ())),
                                      preferred_element_type=f32)
        cos = jax.lax.dot_general(ksel, kf, (((1,), (1,)), ((), ())),
                                  preferred_element_type=f32)
        cos = cos * invnsel * invn
        cos = jnp.where((P > 0.5) | (iota2 == 0), f32(-3.0), cos)

        bits = jax.lax.bitcast_convert_type(cos, i32)
        skey = bits ^ (jax.lax.shift_right_arithmetic(bits, 31)
                       & i32(0x7FFFFFFF))
        skey = (skey & i32(~1023)) | (i32(1023) - iota2)
        skeys.append(skey)
        per_b.append((attn, P, S))

    neg_inf_key = i32(-(2 ** 31) + 1)

    def top32_body(i, carry):
        kw, M = carry
        mx = jnp.max(kw, axis=1, keepdims=True)
        oh = kw == mx
        return jnp.where(oh, neg_inf_key, kw), M + oh.astype(f32)

    skey_all = jnp.concatenate(skeys, axis=0)
    _, M_all = jax.lax.fori_loop(
        0, CLUSTER_K, top32_body,
        (skey_all, jnp.zeros((BB * LEFT, NK), dtype=f32)), unroll=True)

    for bb in range(BB):
        attn, P, S = per_b[bb]
        M = M_all[bb * LEFT:(bb + 1) * LEFT, :]
        extra_w = jnp.where(lane_row == 0, f32(0.0), (1.0 - S) * attn)
        g = jnp.concatenate([P + M * attn, extra_w], axis=0)
        g = g[:, 1:]
        res = jax.lax.dot_general(g, x_ref[bb], (((1,), (0,)), ((), ())),
                                  preferred_element_type=f32)
        out_ref[bb] = res


def kernel(image_features, desired_q, desired_k):
    return pl.pallas_call(
        _kernel,
        grid=(B // BB,),
        in_specs=[
            pl.BlockSpec((BB, 8, C), lambda b: (b, 0, 0)),
            pl.BlockSpec((BB, NK, C), lambda b: (b, 0, 0)),
            pl.BlockSpec((BB, N, C), lambda b: (b, 0, 0)),
        ],
        out_specs=pl.BlockSpec((BB, LEFT + 1, C), lambda b: (b, 0, 0)),
        out_shape=jax.ShapeDtypeStruct((B, LEFT + 1, C), jnp.float32),
        compiler_params=pltpu.CompilerParams(
            dimension_semantics=("arbitrary",)),
    )(desired_q, desired_k, image_features)
